# baseline (speedup 1.0000x reference)
import functools

import jax
import jax.numpy as jnp
from jax.experimental import pallas as pl
from jax.experimental.pallas import tpu as pltpu

_N = 4096
_NG = 6
_HID = 128
_TI = 128
_SJ = 32


def _occ_kernel(xj_ref, yj_ref, oiT_ref, w_ref, b_ref, out_ref):
    t = pl.program_id(0)
    xi = oiT_ref[0:1, :]
    yi = oiT_ref[1:2, :]
    i_ids = t * _TI + jax.lax.broadcasted_iota(jnp.int32, (1, _TI), 1)
    j_iota = jax.lax.broadcasted_iota(jnp.int32, (_SJ, 1), 0)

    def body(c, accs):
        acc_lo, acc_hi = accs
        base = c * _SJ
        sx = xj_ref[pl.ds(base, _SJ), :]
        sy = yj_ref[pl.ds(base, _SJ), :]
        relx = (sx - xi) * 2.0 + 3.0
        rely = (sy - yi) * 2.0 + 3.0
        xbf = jnp.floor(relx)
        ybf = jnp.floor(rely)
        vx = (xbf >= 0.0) & (xbf <= 5.0)
        vy = (ybf >= 0.0) & (ybf <= 5.0)
        ne = (base + j_iota) != i_ids
        valid = vx & vy & ne
        ge3 = xbf >= 3.0
        xmf = jnp.where(ge3, xbf - 3.0, xbf)
        amt = jnp.clip((xmf * 6.0 + ybf).astype(jnp.int32), 0, 17)
        vv = jnp.where(valid, jnp.int32(1) << amt, 0)
        return (acc_lo | jnp.where(ge3, 0, vv),
                acc_hi | jnp.where(ge3, vv, 0))

    zero = jnp.zeros((_SJ, _TI), jnp.int32)
    acc_lo, acc_hi = jax.lax.fori_loop(0, _N // _SJ, body, (zero, zero))

    occ_rows = []
    for acc in (acc_lo, acc_hi):
        a = acc
        s = _SJ
        while s > 8:
            h = s // 2
            a = a[:h] | a[h:s]
            s = h
        for bit_idx in range(18):
            bit = (a >> bit_idx) & 1
            occ_rows.append(
                jnp.max(bit, axis=0, keepdims=True).astype(jnp.float32))
    occT = jnp.concatenate(occ_rows, axis=0)

    out = jax.lax.dot_general(
        occT, w_ref[...],
        dimension_numbers=(((0,), (1,)), ((), ())),
        preferred_element_type=jnp.float32)
    out_ref[...] = out + b_ref[...]


@functools.partial(jax.jit, static_argnames=())
def kernel(h, c, obs, W, b):
    del h, c
    obs = obs.astype(jnp.float32)
    xj = obs[:, 0:1]
    yj = obs[:, 1:2]
    oiT = jnp.concatenate(
        [obs.T, jnp.zeros((6, _N), jnp.float32)], axis=0)
    b2 = b.reshape(1, _HID).astype(jnp.float32)

    grid = (_N // _TI,)
    out = pl.pallas_call(
        _occ_kernel,
        grid=grid,
        in_specs=[
            pl.BlockSpec((_N, 1), lambda t: (0, 0)),
            pl.BlockSpec((_N, 1), lambda t: (0, 0)),
            pl.BlockSpec((8, _TI), lambda t: (0, t)),
            pl.BlockSpec((_HID, _NG * _NG), lambda t: (0, 0)),
            pl.BlockSpec((1, _HID), lambda t: (0, 0)),
        ],
        out_specs=pl.BlockSpec((_TI, _HID), lambda t: (t, 0)),
        out_shape=jax.ShapeDtypeStruct((_N, _HID), jnp.float32),
        compiler_params=pltpu.CompilerParams(
            dimension_semantics=("parallel",)),
    )(xj, yj, oiT, W.astype(jnp.float32), b2)
    return out


# speedup vs baseline: 3.1765x; 3.1765x over previous
import functools

import jax
import jax.numpy as jnp
from jax.experimental import pallas as pl
from jax.experimental.pallas import tpu as pltpu

_N = 4096
_NG = 6
_HID = 128
_TI = 128
_SJ = 512


def _occ_kernel(xj_ref, yj_ref, oiT_ref, w_ref, b_ref, out_ref):
    t = pl.program_id(0)
    xi = oiT_ref[0:1, :]
    yi = oiT_ref[1:2, :]
    i_ids = t * _TI + jax.lax.broadcasted_iota(jnp.int32, (1, _TI), 1)
    j_iota = jax.lax.broadcasted_iota(jnp.int32, (_SJ, 1), 0)

    def body(c, accs):
        acc_lo, acc_hi = accs
        base = c * _SJ
        sx = xj_ref[pl.ds(base, _SJ), :]
        sy = yj_ref[pl.ds(base, _SJ), :]
        relx = (sx - xi) * 2.0 + 3.0
        rely = (sy - yi) * 2.0 + 3.0
        xbf = jnp.floor(relx)
        ybf = jnp.floor(rely)
        vx = (xbf >= 0.0) & (xbf <= 5.0)
        vy = (ybf >= 0.0) & (ybf <= 5.0)
        ne = (base + j_iota) != i_ids
        valid = vx & vy & ne
        ge3 = xbf >= 3.0
        xmf = jnp.where(ge3, xbf - 3.0, xbf)
        amt = jnp.clip((xmf * 6.0 + ybf).astype(jnp.int32), 0, 17)
        vv = jnp.where(valid, jnp.int32(1) << amt, 0)
        return (acc_lo | jnp.where(ge3, 0, vv),
                acc_hi | jnp.where(ge3, vv, 0))

    zero = jnp.zeros((_SJ, _TI), jnp.int32)
    acc_lo, acc_hi = jax.lax.fori_loop(0, _N // _SJ, body, (zero, zero))

    occ_rows = []
    for acc in (acc_lo, acc_hi):
        a = acc
        s = _SJ
        while s > 8:
            h = s // 2
            a = a[:h] | a[h:s]
            s = h
        for bit_idx in range(18):
            bit = (a >> bit_idx) & 1
            occ_rows.append(
                jnp.max(bit, axis=0, keepdims=True).astype(jnp.float32))
    occT = jnp.concatenate(occ_rows, axis=0)

    out = jax.lax.dot_general(
        occT, w_ref[...],
        dimension_numbers=(((0,), (1,)), ((), ())),
        preferred_element_type=jnp.float32)
    out_ref[...] = out + b_ref[...]


@functools.partial(jax.jit, static_argnames=())
def kernel(h, c, obs, W, b):
    del h, c
    obs = obs.astype(jnp.float32)
    xj = obs[:, 0:1]
    yj = obs[:, 1:2]
    oiT = jnp.concatenate(
        [obs.T, jnp.zeros((6, _N), jnp.float32)], axis=0)
    b2 = b.reshape(1, _HID).astype(jnp.float32)

    grid = (_N // _TI,)
    out = pl.pallas_call(
        _occ_kernel,
        grid=grid,
        in_specs=[
            pl.BlockSpec((_N, 1), lambda t: (0, 0)),
            pl.BlockSpec((_N, 1), lambda t: (0, 0)),
            pl.BlockSpec((8, _TI), lambda t: (0, t)),
            pl.BlockSpec((_HID, _NG * _NG), lambda t: (0, 0)),
            pl.BlockSpec((1, _HID), lambda t: (0, 0)),
        ],
        out_specs=pl.BlockSpec((_TI, _HID), lambda t: (t, 0)),
        out_shape=jax.ShapeDtypeStruct((_N, _HID), jnp.float32),
        compiler_params=pltpu.CompilerParams(
            dimension_semantics=("parallel",)),
    )(xj, yj, oiT, W.astype(jnp.float32), b2)
    return out


# speedup vs baseline: 3.3884x; 1.0667x over previous
import functools

import jax
import jax.numpy as jnp
from jax.experimental import pallas as pl
from jax.experimental.pallas import tpu as pltpu

_N = 4096
_NG = 6
_HID = 128
_TI = 128
_SJ = 512


def _occ_kernel(xj_ref, yj_ref, oiT_ref, w_ref, b_ref, out_ref):
    t = pl.program_id(0)
    xi = oiT_ref[0:1, :]
    yi = oiT_ref[1:2, :]
    i_ids = t * _TI + jax.lax.broadcasted_iota(jnp.int32, (1, _TI), 1)
    j_iota = jax.lax.broadcasted_iota(jnp.int32, (_SJ, 1), 0)

    acc_lo = jnp.zeros((_SJ, _TI), jnp.int32)
    acc_hi = jnp.zeros((_SJ, _TI), jnp.int32)
    for c in range(_N // _SJ):
        base = c * _SJ
        sx = xj_ref[base:base + _SJ, :]
        sy = yj_ref[base:base + _SJ, :]
        relx = (sx - xi) * 2.0 + 3.0
        rely = (sy - yi) * 2.0 + 3.0
        xbf = jnp.floor(relx)
        ybf = jnp.floor(rely)
        vx = (xbf >= 0.0) & (xbf <= 5.0)
        vy = (ybf >= 0.0) & (ybf <= 5.0)
        ne = (base + j_iota) != i_ids
        valid = vx & vy & ne
        ge3 = xbf >= 3.0
        xmf = jnp.where(ge3, xbf - 3.0, xbf)
        amt = jnp.clip((xmf * 6.0 + ybf).astype(jnp.int32), 0, 17)
        vv = jnp.where(valid, jnp.int32(1) << amt, 0)
        acc_lo = acc_lo | jnp.where(ge3, 0, vv)
        acc_hi = acc_hi | jnp.where(ge3, vv, 0)

    occ_rows = []
    for acc in (acc_lo, acc_hi):
        a = acc
        s = _SJ
        while s > 8:
            h = s // 2
            a = a[:h] | a[h:s]
            s = h
        for bit_idx in range(18):
            bit = (a >> bit_idx) & 1
            occ_rows.append(
                jnp.max(bit, axis=0, keepdims=True).astype(jnp.float32))
    occT = jnp.concatenate(occ_rows, axis=0)

    out = jax.lax.dot_general(
        occT, w_ref[...],
        dimension_numbers=(((0,), (1,)), ((), ())),
        preferred_element_type=jnp.float32)
    out_ref[...] = out + b_ref[...]


@functools.partial(jax.jit, static_argnames=())
def kernel(h, c, obs, W, b):
    del h, c
    obs = obs.astype(jnp.float32)
    xj = jnp.broadcast_to(obs[:, 0:1], (_N, _TI))
    yj = jnp.broadcast_to(obs[:, 1:2], (_N, _TI))
    oiT = jnp.concatenate(
        [obs.T, jnp.zeros((6, _N), jnp.float32)], axis=0)
    b2 = b.reshape(1, _HID).astype(jnp.float32)

    grid = (_N // _TI,)
    out = pl.pallas_call(
        _occ_kernel,
        grid=grid,
        in_specs=[
            pl.BlockSpec((_N, _TI), lambda t: (0, 0)),
            pl.BlockSpec((_N, _TI), lambda t: (0, 0)),
            pl.BlockSpec((8, _TI), lambda t: (0, t)),
            pl.BlockSpec((_HID, _NG * _NG), lambda t: (0, 0)),
            pl.BlockSpec((1, _HID), lambda t: (0, 0)),
        ],
        out_specs=pl.BlockSpec((_TI, _HID), lambda t: (t, 0)),
        out_shape=jax.ShapeDtypeStruct((_N, _HID), jnp.float32),
        compiler_params=pltpu.CompilerParams(
            dimension_semantics=("parallel",)),
    )(xj, yj, oiT, W.astype(jnp.float32), b2)
    return out


# speedup vs baseline: 4.8426x; 1.4292x over previous
import functools

import jax
import jax.numpy as jnp
from jax.experimental import pallas as pl
from jax.experimental.pallas import tpu as pltpu

_N = 4096
_NG = 6
_HID = 128
_TI = 128
_SJ = 64


def _occ_kernel(xj_ref, yj_ref, oiT_ref, w_ref, b_ref, out_ref):
    t = pl.program_id(0)
    xi = oiT_ref[0:1, :]
    yi = oiT_ref[1:2, :]
    i_ids = t * _TI + jax.lax.broadcasted_iota(jnp.int32, (1, _TI), 1)
    j_iota = jax.lax.broadcasted_iota(jnp.int32, (_SJ, 1), 0)

    acc_lo = jnp.zeros((_SJ, _TI), jnp.int32)
    acc_hi = jnp.zeros((_SJ, _TI), jnp.int32)
    for c in range(_N // _SJ):
        base = c * _SJ
        sx = xj_ref[base:base + _SJ, :]
        sy = yj_ref[base:base + _SJ, :]
        relx = (sx - xi) * 2.0 + 3.0
        rely = (sy - yi) * 2.0 + 3.0
        xbf = jnp.floor(relx)
        ybf = jnp.floor(rely)
        inr = ((jnp.minimum(xbf, ybf) >= 0.0)
               & (jnp.maximum(xbf, ybf) <= 5.0))
        ne = (base + j_iota) != i_ids
        valid = inr & ne
        ci = (xbf * 6.0 + ybf).astype(jnp.int32)
        v32 = jnp.int32(1) << (ci & 31)
        vv = jnp.where(valid, v32, 0)
        is_lo = ci < 32
        acc_lo = acc_lo | jnp.where(is_lo, vv, 0)
        acc_hi = acc_hi | jnp.where(is_lo, 0, vv)

    occ_rows = []
    for acc in (acc_lo, acc_hi):
        a = acc
        s = _SJ
        while s > 8:
            h = s // 2
            a = a[:h] | a[h:s]
            s = h
        nbits = 32 if acc is acc_lo else 4
        for bit_idx in range(nbits):
            bit = (a >> bit_idx) & 1
            occ_rows.append(
                jnp.max(bit, axis=0, keepdims=True).astype(jnp.float32))
    occT = jnp.concatenate(occ_rows, axis=0)

    out = jax.lax.dot_general(
        occT, w_ref[...],
        dimension_numbers=(((0,), (1,)), ((), ())),
        preferred_element_type=jnp.float32)
    out_ref[...] = out + b_ref[...]


@functools.partial(jax.jit, static_argnames=())
def kernel(h, c, obs, W, b):
    del h, c
    obs = obs.astype(jnp.float32)
    xj = jnp.broadcast_to(obs[:, 0:1], (_N, _TI))
    yj = jnp.broadcast_to(obs[:, 1:2], (_N, _TI))
    oiT = jnp.concatenate(
        [obs.T, jnp.zeros((6, _N), jnp.float32)], axis=0)
    b2 = b.reshape(1, _HID).astype(jnp.float32)

    grid = (_N // _TI,)
    out = pl.pallas_call(
        _occ_kernel,
        grid=grid,
        in_specs=[
            pl.BlockSpec((_N, _TI), lambda t: (0, 0)),
            pl.BlockSpec((_N, _TI), lambda t: (0, 0)),
            pl.BlockSpec((8, _TI), lambda t: (0, t)),
            pl.BlockSpec((_HID, _NG * _NG), lambda t: (0, 0)),
            pl.BlockSpec((1, _HID), lambda t: (0, 0)),
        ],
        out_specs=pl.BlockSpec((_TI, _HID), lambda t: (t, 0)),
        out_shape=jax.ShapeDtypeStruct((_N, _HID), jnp.float32),
        compiler_params=pltpu.CompilerParams(
            dimension_semantics=("parallel",)),
    )(xj, yj, oiT, W.astype(jnp.float32), b2)
    return out


# speedup vs baseline: 4.8466x; 1.0008x over previous
import functools

import jax
import jax.numpy as jnp
from jax.experimental import pallas as pl
from jax.experimental.pallas import tpu as pltpu

_N = 4096
_NG = 6
_HID = 128
_TI = 128
_SJ = 64


def _occ_kernel(xj_ref, yj_ref, oiT_ref, w_ref, b_ref, out_ref):
    t = pl.program_id(0)
    xi = oiT_ref[0:1, :]
    yi = oiT_ref[1:2, :]
    i_ids = t * _TI + jax.lax.broadcasted_iota(jnp.int32, (1, _TI), 1)
    j_iota = jax.lax.broadcasted_iota(jnp.int32, (_SJ, 1), 0)

    acc_lo = jnp.zeros((_SJ, _TI), jnp.int32)
    acc_hi = jnp.zeros((_SJ, _TI), jnp.int32)
    for c in range(_N // _SJ):
        base = c * _SJ
        sx = xj_ref[base:base + _SJ, :]
        sy = yj_ref[base:base + _SJ, :]
        relx = (sx - xi) * 2.0 + 3.0
        rely = (sy - yi) * 2.0 + 3.0
        inr = ((jnp.minimum(relx, rely) >= 0.0)
               & (jnp.maximum(relx, rely) < 6.0))
        ne = (base + j_iota) != i_ids
        valid = inr & ne
        ci = relx.astype(jnp.int32) * 6 + rely.astype(jnp.int32)
        v32 = jnp.int32(1) << (ci & 31)
        vv = jnp.where(valid, v32, 0)
        lo_add = jnp.where(ci < 32, vv, 0)
        acc_lo = acc_lo | lo_add
        acc_hi = acc_hi | (vv ^ lo_add)

    occ_rows = []
    for acc in (acc_lo, acc_hi):
        a = acc
        s = _SJ
        while s > 8:
            h = s // 2
            a = a[:h] | a[h:s]
            s = h
        nbits = 32 if acc is acc_lo else 4
        for bit_idx in range(nbits):
            bit = (a >> bit_idx) & 1
            occ_rows.append(
                jnp.max(bit, axis=0, keepdims=True).astype(jnp.float32))
    occT = jnp.concatenate(occ_rows, axis=0)

    out = jax.lax.dot_general(
        occT, w_ref[...],
        dimension_numbers=(((0,), (1,)), ((), ())),
        preferred_element_type=jnp.float32)
    out_ref[...] = out + b_ref[...]


@functools.partial(jax.jit, static_argnames=())
def kernel(h, c, obs, W, b):
    del h, c
    obs = obs.astype(jnp.float32)
    xj = jnp.broadcast_to(obs[:, 0:1], (_N, _TI))
    yj = jnp.broadcast_to(obs[:, 1:2], (_N, _TI))
    oiT = jnp.concatenate(
        [obs.T, jnp.zeros((6, _N), jnp.float32)], axis=0)
    b2 = b.reshape(1, _HID).astype(jnp.float32)

    grid = (_N // _TI,)
    out = pl.pallas_call(
        _occ_kernel,
        grid=grid,
        in_specs=[
            pl.BlockSpec((_N, _TI), lambda t: (0, 0)),
            pl.BlockSpec((_N, _TI), lambda t: (0, 0)),
            pl.BlockSpec((8, _TI), lambda t: (0, t)),
            pl.BlockSpec((_HID, _NG * _NG), lambda t: (0, 0)),
            pl.BlockSpec((1, _HID), lambda t: (0, 0)),
        ],
        out_specs=pl.BlockSpec((_TI, _HID), lambda t: (t, 0)),
        out_shape=jax.ShapeDtypeStruct((_N, _HID), jnp.float32),
        compiler_params=pltpu.CompilerParams(
            dimension_semantics=("parallel",)),
    )(xj, yj, oiT, W.astype(jnp.float32), b2)
    return out


# speedup vs baseline: 5.0860x; 1.0494x over previous
import functools

import jax
import jax.numpy as jnp
from jax.experimental import pallas as pl
from jax.experimental.pallas import tpu as pltpu

_N = 4096
_NG = 6
_HID = 128
_TI = 128
_SJ = 64


def _occ_kernel(xj_ref, yj_ref, oiT_ref, w_ref, b_ref, out_ref):
    t = pl.program_id(0)
    xi = oiT_ref[0:1, :]
    yi = oiT_ref[1:2, :]
    i_ids = t * _TI + jax.lax.broadcasted_iota(jnp.int32, (1, _TI), 1)
    j_iota = jax.lax.broadcasted_iota(jnp.int32, (_SJ, 1), 0)

    acc_lo = jnp.zeros((_SJ, _TI), jnp.int32)
    acc_hi = jnp.zeros((_SJ, _TI), jnp.int32)
    for c in range(_N // _SJ):
        base = c * _SJ
        sx = xj_ref[base:base + _SJ, :]
        sy = yj_ref[base:base + _SJ, :]
        relx = (sx - xi) + 3.0
        rely = (sy - yi) + 3.0
        inr = ((jnp.minimum(relx, rely) >= 0.0)
               & (jnp.maximum(relx, rely) < 6.0))
        ne = (base + j_iota) != i_ids
        valid = inr & ne
        ci = relx.astype(jnp.int32) * 6 + rely.astype(jnp.int32)
        v32 = jnp.int32(1) << (ci & 31)
        vv = jnp.where(valid, v32, 0)
        lo_add = jnp.where(ci < 32, vv, 0)
        acc_lo = acc_lo | lo_add
        acc_hi = acc_hi | (vv ^ lo_add)

    occ_rows = []
    for acc in (acc_lo, acc_hi):
        a = acc
        s = _SJ
        while s > 8:
            h = s // 2
            a = a[:h] | a[h:s]
            s = h
        nbits = 32 if acc is acc_lo else 4
        for bit_idx in range(nbits):
            bit = (a >> bit_idx) & 1
            occ_rows.append(
                jnp.max(bit, axis=0, keepdims=True).astype(jnp.float32))
    occT = jnp.concatenate(occ_rows, axis=0)

    out = jax.lax.dot_general(
        occT, w_ref[...],
        dimension_numbers=(((0,), (1,)), ((), ())),
        preferred_element_type=jnp.float32)
    out_ref[...] = out + b_ref[...]


@functools.partial(jax.jit, static_argnames=())
def kernel(h, c, obs, W, b):
    del h, c
    obs2 = obs.astype(jnp.float32) * 2.0
    xj = jnp.broadcast_to(obs2[:, 0:1], (_N, _TI))
    yj = jnp.broadcast_to(obs2[:, 1:2], (_N, _TI))
    oiT = jnp.concatenate(
        [obs2.T, jnp.zeros((6, _N), jnp.float32)], axis=0)
    b2 = b.reshape(1, _HID).astype(jnp.float32)

    grid = (_N // _TI,)
    out = pl.pallas_call(
        _occ_kernel,
        grid=grid,
        in_specs=[
            pl.BlockSpec((_N, _TI), lambda t: (0, 0)),
            pl.BlockSpec((_N, _TI), lambda t: (0, 0)),
            pl.BlockSpec((8, _TI), lambda t: (0, t)),
            pl.BlockSpec((_HID, _NG * _NG), lambda t: (0, 0)),
            pl.BlockSpec((1, _HID), lambda t: (0, 0)),
        ],
        out_specs=pl.BlockSpec((_TI, _HID), lambda t: (t, 0)),
        out_shape=jax.ShapeDtypeStruct((_N, _HID), jnp.float32),
        compiler_params=pltpu.CompilerParams(
            dimension_semantics=("parallel",)),
    )(xj, yj, oiT, W.astype(jnp.float32), b2)
    return out


# speedup vs baseline: 5.9433x; 1.1686x over previous
import functools

import jax
import jax.numpy as jnp
from jax.experimental import pallas as pl
from jax.experimental.pallas import tpu as pltpu

_N = 4096
_NG = 6
_HID = 128
_TI = 128
_SJ = 64


def _occ_kernel(xj_ref, yj_ref, oiT_ref, w_ref, b_ref, out_ref):
    t = pl.program_id(0)
    xi = oiT_ref[0:1, :]
    yi = oiT_ref[1:2, :]
    i_ids = t * _TI + jax.lax.broadcasted_iota(jnp.int32, (1, _TI), 1)
    j_iota = jax.lax.broadcasted_iota(jnp.int32, (_SJ, 1), 0)

    acc_lo = jnp.zeros((_SJ, _TI), jnp.int32)
    acc_hi = jnp.zeros((_SJ, _TI), jnp.int32)
    for c in range(_N // _SJ):
        base = c * _SJ
        sx = xj_ref[base:base + _SJ, :]
        sy = yj_ref[base:base + _SJ, :]
        relx = (sx - xi) + 3.0
        rely = (sy - yi) + 3.0
        SIX = jnp.uint32(0x40C00000)
        vx = relx.view(jnp.uint32) < SIX
        vy = rely.view(jnp.uint32) < SIX
        ne = (base + j_iota) != i_ids
        valid = (vx & vy) & ne
        ci = (relx.astype(jnp.int32) << 3) | rely.astype(jnp.int32)
        v32 = jnp.int32(1) << (ci & 31)
        vv = jnp.where(valid, v32, 0)
        lo_add = jnp.where(ci < 32, vv, 0)
        acc_lo = acc_lo | lo_add
        acc_hi = acc_hi | (vv ^ lo_add)

    red = []
    for acc in (acc_lo, acc_hi):
        a = acc
        s = _SJ
        while s > 8:
            h = s // 2
            a = a[:h] | a[h:s]
            s = h
        red.append(a)
    occ_rows = []
    for bx in range(_NG):
        for by in range(_NG):
            code = bx * 8 + by
            a, bit_idx = (red[0], code) if code < 32 else (red[1], code - 32)
            bit = (a >> bit_idx) & 1
            occ_rows.append(
                jnp.max(bit, axis=0, keepdims=True).astype(jnp.float32))
    occT = jnp.concatenate(occ_rows, axis=0)

    out = jax.lax.dot_general(
        occT, w_ref[...],
        dimension_numbers=(((0,), (1,)), ((), ())),
        preferred_element_type=jnp.float32)
    out_ref[...] = out + b_ref[...]


@functools.partial(jax.jit, static_argnames=())
def kernel(h, c, obs, W, b):
    del h, c
    obs2 = obs.astype(jnp.float32) * 2.0
    xj = jnp.broadcast_to(obs2[:, 0:1], (_N, _TI))
    yj = jnp.broadcast_to(obs2[:, 1:2], (_N, _TI))
    oiT = jnp.concatenate(
        [obs2.T, jnp.zeros((6, _N), jnp.float32)], axis=0)
    b2 = b.reshape(1, _HID).astype(jnp.float32)

    grid = (_N // _TI,)
    out = pl.pallas_call(
        _occ_kernel,
        grid=grid,
        in_specs=[
            pl.BlockSpec((_N, _TI), lambda t: (0, 0)),
            pl.BlockSpec((_N, _TI), lambda t: (0, 0)),
            pl.BlockSpec((8, _TI), lambda t: (0, t)),
            pl.BlockSpec((_HID, _NG * _NG), lambda t: (0, 0)),
            pl.BlockSpec((1, _HID), lambda t: (0, 0)),
        ],
        out_specs=pl.BlockSpec((_TI, _HID), lambda t: (t, 0)),
        out_shape=jax.ShapeDtypeStruct((_N, _HID), jnp.float32),
        compiler_params=pltpu.CompilerParams(
            dimension_semantics=("parallel",)),
    )(xj, yj, oiT, W.astype(jnp.float32), b2)
    return out


# speedup vs baseline: 6.0878x; 1.0243x over previous
import functools

import jax
import jax.numpy as jnp
from jax.experimental import pallas as pl
from jax.experimental.pallas import tpu as pltpu

_N = 4096
_NG = 6
_HID = 128
_TI = 128
_SJ = 64


def _occ_kernel(xj_ref, yj_ref, oiT_ref, w_ref, b_ref, out_ref):
    t = pl.program_id(0)
    xi = oiT_ref[0:1, :]
    yi = oiT_ref[1:2, :]
    i_ids = t * _TI + jax.lax.broadcasted_iota(jnp.int32, (1, _TI), 1)
    j_iota = jax.lax.broadcasted_iota(jnp.int32, (_SJ, 1), 0)

    acc_lo = jnp.zeros((_SJ, _TI), jnp.int32)
    acc_hi = jnp.zeros((_SJ, _TI), jnp.int32)
    for c in range(_N // _SJ):
        base = c * _SJ
        sx = xj_ref[base:base + _SJ, :]
        sy = yj_ref[base:base + _SJ, :]
        relx = (sx - xi) + 3.0
        rely = (sy - yi) + 3.0
        SIX = jnp.uint32(0x40C00000)
        vx = relx.view(jnp.uint32) < SIX
        vy = rely.view(jnp.uint32) < SIX
        ne = (base + j_iota) != i_ids
        valid = (vx & vy) & ne
        ci = (relx.astype(jnp.int32) << 3) | rely.astype(jnp.int32)
        v32 = jnp.int32(1) << (ci & 31)
        vv = jnp.where(valid, v32, 0)
        lo_add = jnp.where(ci < 32, vv, 0)
        acc_lo = acc_lo | lo_add
        acc_hi = acc_hi | (vv ^ lo_add)

    red = []
    for acc in (acc_lo, acc_hi):
        a = acc
        s = _SJ
        while s > 1:
            h = s // 2
            a = a[:h] | a[h:s]
            s = h
        red.append(a)
    lo_b = jnp.broadcast_to(red[0], (32, _TI))
    hi_b = jnp.broadcast_to(red[1], (16, _TI))
    sh32 = jax.lax.broadcasted_iota(jnp.int32, (32, 1), 0)
    sh16 = jax.lax.broadcasted_iota(jnp.int32, (16, 1), 0)
    b_lo = (lo_b >> sh32) & 1
    b_hi = (hi_b >> sh16) & 1
    occT = jnp.concatenate([b_lo, b_hi], axis=0).astype(jnp.float32)

    out = jax.lax.dot_general(
        occT, w_ref[...],
        dimension_numbers=(((0,), (1,)), ((), ())),
        preferred_element_type=jnp.float32)
    out_ref[...] = out + b_ref[...]


@functools.partial(jax.jit, static_argnames=())
def kernel(h, c, obs, W, b):
    del h, c
    obs2 = obs.astype(jnp.float32) * 2.0
    xj = jnp.broadcast_to(obs2[:, 0:1], (_N, _TI))
    yj = jnp.broadcast_to(obs2[:, 1:2], (_N, _TI))
    oiT = jnp.concatenate(
        [obs2.T, jnp.zeros((6, _N), jnp.float32)], axis=0)
    b2 = b.reshape(1, _HID).astype(jnp.float32)
    cols = jnp.array([bx * 8 + by for bx in range(_NG) for by in range(_NG)],
                     dtype=jnp.int32)
    W48 = jnp.zeros((_HID, 48), jnp.float32).at[:, cols].set(
        W.astype(jnp.float32))

    grid = (_N // _TI,)
    out = pl.pallas_call(
        _occ_kernel,
        grid=grid,
        in_specs=[
            pl.BlockSpec((_N, _TI), lambda t: (0, 0)),
            pl.BlockSpec((_N, _TI), lambda t: (0, 0)),
            pl.BlockSpec((8, _TI), lambda t: (0, t)),
            pl.BlockSpec((_HID, 48), lambda t: (0, 0)),
            pl.BlockSpec((1, _HID), lambda t: (0, 0)),
        ],
        out_specs=pl.BlockSpec((_TI, _HID), lambda t: (t, 0)),
        out_shape=jax.ShapeDtypeStruct((_N, _HID), jnp.float32),
        compiler_params=pltpu.CompilerParams(
            dimension_semantics=("parallel",)),
    )(xj, yj, oiT, W48, b2)
    return out


# speedup vs baseline: 6.5416x; 1.0746x over previous
import functools

import jax
import jax.numpy as jnp
from jax.experimental import pallas as pl
from jax.experimental.pallas import tpu as pltpu

_N = 4096
_NG = 6
_HID = 128
_TI = 128
_TPS = 2
_SJ = 64


def _occ_kernel(xj_ref, yj_ref, oiT_ref, w_ref, b_ref, out_ref):
    t = pl.program_id(0)
    SIX = jnp.uint32(0x40C00000)
    j_iota = jax.lax.broadcasted_iota(jnp.int32, (_SJ, 1), 0)
    l_iota = jax.lax.broadcasted_iota(jnp.int32, (1, _TI), 1)

    xis, yis, i_idss = [], [], []
    for k in range(_TPS):
        xis.append(oiT_ref[0:1, k * _TI:(k + 1) * _TI])
        yis.append(oiT_ref[1:2, k * _TI:(k + 1) * _TI])
        i_idss.append((t * _TPS + k) * _TI + l_iota)

    accs = [[jnp.zeros((_SJ, _TI), jnp.int32) for _ in range(2)]
            for _ in range(_TPS)]
    for c in range(_N // _SJ):
        base = c * _SJ
        sx = xj_ref[base:base + _SJ, :]
        sy = yj_ref[base:base + _SJ, :]
        for k in range(_TPS):
            relx = (sx - xis[k]) + 3.0
            rely = (sy - yis[k]) + 3.0
            vx = relx.view(jnp.uint32) < SIX
            vy = rely.view(jnp.uint32) < SIX
            ne = (base + j_iota) != i_idss[k]
            valid = (vx & vy) & ne
            ci = (relx.astype(jnp.int32) << 3) | rely.astype(jnp.int32)
            v32 = jnp.int32(1) << (ci & 31)
            vv = jnp.where(valid, v32, 0)
            lo_add = jnp.where(ci < 32, vv, 0)
            accs[k][0] = accs[k][0] | lo_add
            accs[k][1] = accs[k][1] | (vv ^ lo_add)

    sh32 = jax.lax.broadcasted_iota(jnp.int32, (32, 1), 0)
    sh16 = jax.lax.broadcasted_iota(jnp.int32, (16, 1), 0)
    for k in range(_TPS):
        red = []
        for acc in accs[k]:
            a = acc
            s = _SJ
            while s > 1:
                h = s // 2
                a = a[:h] | a[h:s]
                s = h
            red.append(a)
        lo_b = jnp.broadcast_to(red[0], (32, _TI))
        hi_b = jnp.broadcast_to(red[1], (16, _TI))
        b_lo = (lo_b >> sh32) & 1
        b_hi = (hi_b >> sh16) & 1
        occT = jnp.concatenate([b_lo, b_hi], axis=0).astype(jnp.float32)

        out = jax.lax.dot_general(
            occT, w_ref[...],
            dimension_numbers=(((0,), (1,)), ((), ())),
            preferred_element_type=jnp.float32)
        out_ref[k * _TI:(k + 1) * _TI, :] = out + b_ref[...]


@functools.partial(jax.jit, static_argnames=())
def kernel(h, c, obs, W, b):
    del h, c
    obs2 = obs.astype(jnp.float32) * 2.0
    xj = jnp.broadcast_to(obs2[:, 0:1], (_N, _TI))
    yj = jnp.broadcast_to(obs2[:, 1:2], (_N, _TI))
    oiT = jnp.concatenate(
        [obs2.T, jnp.zeros((6, _N), jnp.float32)], axis=0)
    b2 = b.reshape(1, _HID).astype(jnp.float32)
    cols = jnp.array([bx * 8 + by for bx in range(_NG) for by in range(_NG)],
                     dtype=jnp.int32)
    W48 = jnp.zeros((_HID, 48), jnp.float32).at[:, cols].set(
        W.astype(jnp.float32))

    grid = (_N // (_TI * _TPS),)
    out = pl.pallas_call(
        _occ_kernel,
        grid=grid,
        in_specs=[
            pl.BlockSpec((_N, _TI), lambda t: (0, 0)),
            pl.BlockSpec((_N, _TI), lambda t: (0, 0)),
            pl.BlockSpec((8, _TPS * _TI), lambda t: (0, t)),
            pl.BlockSpec((_HID, 48), lambda t: (0, 0)),
            pl.BlockSpec((1, _HID), lambda t: (0, 0)),
        ],
        out_specs=pl.BlockSpec((_TPS * _TI, _HID), lambda t: (t, 0)),
        out_shape=jax.ShapeDtypeStruct((_N, _HID), jnp.float32),
        compiler_params=pltpu.CompilerParams(
            dimension_semantics=("parallel",)),
    )(xj, yj, oiT, W48, b2)
    return out


# speedup vs baseline: 6.8304x; 1.0441x over previous
import functools

import jax
import jax.numpy as jnp
from jax.experimental import pallas as pl
from jax.experimental.pallas import tpu as pltpu

_N = 4096
_NG = 6
_HID = 128
_TI = 128
_TPS = 4
_SJ = 64


def _occ_kernel(xj_ref, yj_ref, oiT_ref, w_ref, b_ref, out_ref):
    t = pl.program_id(0)
    SIX = jnp.uint32(0x40C00000)
    j_iota = jax.lax.broadcasted_iota(jnp.int32, (_SJ, 1), 0)
    l_iota = jax.lax.broadcasted_iota(jnp.int32, (1, _TI), 1)

    xis, yis, i_idss = [], [], []
    for k in range(_TPS):
        xis.append(oiT_ref[0:1, k * _TI:(k + 1) * _TI])
        yis.append(oiT_ref[1:2, k * _TI:(k + 1) * _TI])
        i_idss.append((t * _TPS + k) * _TI + l_iota)

    accs = [[jnp.zeros((_SJ, _TI), jnp.int32) for _ in range(2)]
            for _ in range(_TPS)]
    for c in range(_N // _SJ):
        base = c * _SJ
        sx = xj_ref[base:base + _SJ, :]
        sy = yj_ref[base:base + _SJ, :]
        for k in range(_TPS):
            relx = (sx - xis[k]) + 3.0
            rely = (sy - yis[k]) + 3.0
            vx = relx.view(jnp.uint32) < SIX
            vy = rely.view(jnp.uint32) < SIX
            ne = (base + j_iota) != i_idss[k]
            valid = (vx & vy) & ne
            ci = (relx.astype(jnp.int32) << 3) | rely.astype(jnp.int32)
            v32 = jnp.int32(1) << (ci & 31)
            vv = jnp.where(valid, v32, 0)
            lo_add = jnp.where(ci < 32, vv, 0)
            accs[k][0] = accs[k][0] | lo_add
            accs[k][1] = accs[k][1] | (vv ^ lo_add)

    sh32 = jax.lax.broadcasted_iota(jnp.int32, (32, 1), 0)
    sh16 = jax.lax.broadcasted_iota(jnp.int32, (16, 1), 0)
    for k in range(_TPS):
        red = []
        for acc in accs[k]:
            a = acc
            s = _SJ
            while s > 1:
                h = s // 2
                a = a[:h] | a[h:s]
                s = h
            red.append(a)
        lo_b = jnp.broadcast_to(red[0], (32, _TI))
        hi_b = jnp.broadcast_to(red[1], (16, _TI))
        b_lo = (lo_b >> sh32) & 1
        b_hi = (hi_b >> sh16) & 1
        occT = jnp.concatenate([b_lo, b_hi], axis=0).astype(jnp.float32)

        out = jax.lax.dot_general(
            occT, w_ref[...],
            dimension_numbers=(((0,), (1,)), ((), ())),
            preferred_element_type=jnp.float32)
        out_ref[k * _TI:(k + 1) * _TI, :] = out + b_ref[...]


@functools.partial(jax.jit, static_argnames=())
def kernel(h, c, obs, W, b):
    del h, c
    obs2 = obs.astype(jnp.float32) * 2.0
    xj = jnp.broadcast_to(obs2[:, 0:1], (_N, _TI))
    yj = jnp.broadcast_to(obs2[:, 1:2], (_N, _TI))
    oiT = jnp.concatenate(
        [obs2.T, jnp.zeros((6, _N), jnp.float32)], axis=0)
    b2 = b.reshape(1, _HID).astype(jnp.float32)
    cols = jnp.array([bx * 8 + by for bx in range(_NG) for by in range(_NG)],
                     dtype=jnp.int32)
    W48 = jnp.zeros((_HID, 48), jnp.float32).at[:, cols].set(
        W.astype(jnp.float32))

    grid = (_N // (_TI * _TPS),)
    out = pl.pallas_call(
        _occ_kernel,
        grid=grid,
        in_specs=[
            pl.BlockSpec((_N, _TI), lambda t: (0, 0)),
            pl.BlockSpec((_N, _TI), lambda t: (0, 0)),
            pl.BlockSpec((8, _TPS * _TI), lambda t: (0, t)),
            pl.BlockSpec((_HID, 48), lambda t: (0, 0)),
            pl.BlockSpec((1, _HID), lambda t: (0, 0)),
        ],
        out_specs=pl.BlockSpec((_TPS * _TI, _HID), lambda t: (t, 0)),
        out_shape=jax.ShapeDtypeStruct((_N, _HID), jnp.float32),
        compiler_params=pltpu.CompilerParams(
            dimension_semantics=("parallel",)),
    )(xj, yj, oiT, W48, b2)
    return out


# speedup vs baseline: 7.0507x; 1.0323x over previous
import functools

import jax
import jax.numpy as jnp
from jax.experimental import pallas as pl
from jax.experimental.pallas import tpu as pltpu

_N = 4096
_NG = 6
_HID = 128
_TI = 128
_TPS = 8
_SJ = 64


def _occ_kernel(xj_ref, yj_ref, oiT_ref, w_ref, b_ref, out_ref):
    t = pl.program_id(0)
    SIX = jnp.uint32(0x40C00000)
    j_iota = jax.lax.broadcasted_iota(jnp.int32, (_SJ, 1), 0)
    l_iota = jax.lax.broadcasted_iota(jnp.int32, (1, _TI), 1)

    xis, yis, i_idss = [], [], []
    for k in range(_TPS):
        xis.append(oiT_ref[0:1, k * _TI:(k + 1) * _TI])
        yis.append(oiT_ref[1:2, k * _TI:(k + 1) * _TI])
        i_idss.append((t * _TPS + k) * _TI + l_iota)

    accs = [[jnp.zeros((_SJ, _TI), jnp.int32) for _ in range(2)]
            for _ in range(_TPS)]
    for c in range(_N // _SJ):
        base = c * _SJ
        sx = xj_ref[base:base + _SJ, :]
        sy = yj_ref[base:base + _SJ, :]
        for k in range(_TPS):
            relx = (sx - xis[k]) + 3.0
            rely = (sy - yis[k]) + 3.0
            vx = relx.view(jnp.uint32) < SIX
            vy = rely.view(jnp.uint32) < SIX
            ne = (base + j_iota) != i_idss[k]
            valid = (vx & vy) & ne
            ci = (relx.astype(jnp.int32) << 3) | rely.astype(jnp.int32)
            v32 = jnp.int32(1) << (ci & 31)
            vv = jnp.where(valid, v32, 0)
            lo_add = jnp.where(ci < 32, vv, 0)
            accs[k][0] = accs[k][0] | lo_add
            accs[k][1] = accs[k][1] | (vv ^ lo_add)

    sh32 = jax.lax.broadcasted_iota(jnp.int32, (32, 1), 0)
    sh16 = jax.lax.broadcasted_iota(jnp.int32, (16, 1), 0)
    for k in range(_TPS):
        red = []
        for acc in accs[k]:
            a = acc
            s = _SJ
            while s > 1:
                h = s // 2
                a = a[:h] | a[h:s]
                s = h
            red.append(a)
        lo_b = jnp.broadcast_to(red[0], (32, _TI))
        hi_b = jnp.broadcast_to(red[1], (16, _TI))
        b_lo = (lo_b >> sh32) & 1
        b_hi = (hi_b >> sh16) & 1
        occT = jnp.concatenate([b_lo, b_hi], axis=0).astype(jnp.float32)

        out = jax.lax.dot_general(
            occT, w_ref[...],
            dimension_numbers=(((0,), (1,)), ((), ())),
            preferred_element_type=jnp.float32)
        out_ref[k * _TI:(k + 1) * _TI, :] = out + b_ref[...]


@functools.partial(jax.jit, static_argnames=())
def kernel(h, c, obs, W, b):
    del h, c
    obs2 = obs.astype(jnp.float32) * 2.0
    xj = jnp.broadcast_to(obs2[:, 0:1], (_N, _TI))
    yj = jnp.broadcast_to(obs2[:, 1:2], (_N, _TI))
    oiT = jnp.concatenate(
        [obs2.T, jnp.zeros((6, _N), jnp.float32)], axis=0)
    b2 = b.reshape(1, _HID).astype(jnp.float32)
    cols = jnp.array([bx * 8 + by for bx in range(_NG) for by in range(_NG)],
                     dtype=jnp.int32)
    W48 = jnp.zeros((_HID, 48), jnp.float32).at[:, cols].set(
        W.astype(jnp.float32))

    grid = (_N // (_TI * _TPS),)
    out = pl.pallas_call(
        _occ_kernel,
        grid=grid,
        in_specs=[
            pl.BlockSpec((_N, _TI), lambda t: (0, 0)),
            pl.BlockSpec((_N, _TI), lambda t: (0, 0)),
            pl.BlockSpec((8, _TPS * _TI), lambda t: (0, t)),
            pl.BlockSpec((_HID, 48), lambda t: (0, 0)),
            pl.BlockSpec((1, _HID), lambda t: (0, 0)),
        ],
        out_specs=pl.BlockSpec((_TPS * _TI, _HID), lambda t: (t, 0)),
        out_shape=jax.ShapeDtypeStruct((_N, _HID), jnp.float32),
        compiler_params=pltpu.CompilerParams(
            dimension_semantics=("parallel",)),
    )(xj, yj, oiT, W48, b2)
    return out


# speedup vs baseline: 7.1259x; 1.0107x over previous
import functools

import jax
import jax.numpy as jnp
from jax.experimental import pallas as pl
from jax.experimental.pallas import tpu as pltpu

_N = 4096
_NG = 6
_HID = 128
_TI = 128
_TPS = 16
_SJ = 64


def _occ_kernel(xj_ref, yj_ref, oiT_ref, w_ref, b_ref, out_ref):
    t = pl.program_id(0)
    SIX = jnp.uint32(0x40C00000)
    j_iota = jax.lax.broadcasted_iota(jnp.int32, (_SJ, 1), 0)
    l_iota = jax.lax.broadcasted_iota(jnp.int32, (1, _TI), 1)

    xis, yis, i_idss = [], [], []
    for k in range(_TPS):
        xis.append(oiT_ref[0:1, k * _TI:(k + 1) * _TI])
        yis.append(oiT_ref[1:2, k * _TI:(k + 1) * _TI])
        i_idss.append((t * _TPS + k) * _TI + l_iota)

    accs = [[jnp.zeros((_SJ, _TI), jnp.int32) for _ in range(2)]
            for _ in range(_TPS)]
    for c in range(_N // _SJ):
        base = c * _SJ
        sx = xj_ref[base:base + _SJ, :]
        sy = yj_ref[base:base + _SJ, :]
        for k in range(_TPS):
            relx = (sx - xis[k]) + 3.0
            rely = (sy - yis[k]) + 3.0
            vx = relx.view(jnp.uint32) < SIX
            vy = rely.view(jnp.uint32) < SIX
            ne = (base + j_iota) != i_idss[k]
            valid = (vx & vy) & ne
            ci = (relx.astype(jnp.int32) << 3) | rely.astype(jnp.int32)
            v32 = jnp.int32(1) << (ci & 31)
            vv = jnp.where(valid, v32, 0)
            lo_add = jnp.where(ci < 32, vv, 0)
            accs[k][0] = accs[k][0] | lo_add
            accs[k][1] = accs[k][1] | (vv ^ lo_add)

    sh32 = jax.lax.broadcasted_iota(jnp.int32, (32, 1), 0)
    sh16 = jax.lax.broadcasted_iota(jnp.int32, (16, 1), 0)
    for k in range(_TPS):
        red = []
        for acc in accs[k]:
            a = acc
            s = _SJ
            while s > 1:
                h = s // 2
                a = a[:h] | a[h:s]
                s = h
            red.append(a)
        lo_b = jnp.broadcast_to(red[0], (32, _TI))
        hi_b = jnp.broadcast_to(red[1], (16, _TI))
        b_lo = (lo_b >> sh32) & 1
        b_hi = (hi_b >> sh16) & 1
        occT = jnp.concatenate([b_lo, b_hi], axis=0).astype(jnp.float32)

        out = jax.lax.dot_general(
            occT, w_ref[...],
            dimension_numbers=(((0,), (1,)), ((), ())),
            preferred_element_type=jnp.float32)
        out_ref[k * _TI:(k + 1) * _TI, :] = out + b_ref[...]


@functools.partial(jax.jit, static_argnames=())
def kernel(h, c, obs, W, b):
    del h, c
    obs2 = obs.astype(jnp.float32) * 2.0
    xj = jnp.broadcast_to(obs2[:, 0:1], (_N, _TI))
    yj = jnp.broadcast_to(obs2[:, 1:2], (_N, _TI))
    oiT = jnp.concatenate(
        [obs2.T, jnp.zeros((6, _N), jnp.float32)], axis=0)
    b2 = b.reshape(1, _HID).astype(jnp.float32)
    cols = jnp.array([bx * 8 + by for bx in range(_NG) for by in range(_NG)],
                     dtype=jnp.int32)
    W48 = jnp.zeros((_HID, 48), jnp.float32).at[:, cols].set(
        W.astype(jnp.float32))

    grid = (_N // (_TI * _TPS),)
    out = pl.pallas_call(
        _occ_kernel,
        grid=grid,
        in_specs=[
            pl.BlockSpec((_N, _TI), lambda t: (0, 0)),
            pl.BlockSpec((_N, _TI), lambda t: (0, 0)),
            pl.BlockSpec((8, _TPS * _TI), lambda t: (0, t)),
            pl.BlockSpec((_HID, 48), lambda t: (0, 0)),
            pl.BlockSpec((1, _HID), lambda t: (0, 0)),
        ],
        out_specs=pl.BlockSpec((_TPS * _TI, _HID), lambda t: (t, 0)),
        out_shape=jax.ShapeDtypeStruct((_N, _HID), jnp.float32),
        compiler_params=pltpu.CompilerParams(
            dimension_semantics=("parallel",)),
    )(xj, yj, oiT, W48, b2)
    return out


# speedup vs baseline: 7.2588x; 1.0186x over previous
import functools

import jax
import jax.numpy as jnp
from jax.experimental import pallas as pl
from jax.experimental.pallas import tpu as pltpu

_N = 4096
_NG = 6
_HID = 128
_TI = 128
_TPS = 32
_SJ = 64


def _occ_kernel(xj_ref, yj_ref, oiT_ref, w_ref, b_ref, out_ref):
    t = pl.program_id(0)
    SIX = jnp.uint32(0x40C00000)
    j_iota = jax.lax.broadcasted_iota(jnp.int32, (_SJ, 1), 0)
    l_iota = jax.lax.broadcasted_iota(jnp.int32, (1, _TI), 1)

    xis, yis, i_idss = [], [], []
    for k in range(_TPS):
        xis.append(oiT_ref[0:1, k * _TI:(k + 1) * _TI])
        yis.append(oiT_ref[1:2, k * _TI:(k + 1) * _TI])
        i_idss.append((t * _TPS + k) * _TI + l_iota)

    accs = [[jnp.zeros((_SJ, _TI), jnp.int32) for _ in range(2)]
            for _ in range(_TPS)]
    for c in range(_N // _SJ):
        base = c * _SJ
        sx = xj_ref[base:base + _SJ, :]
        sy = yj_ref[base:base + _SJ, :]
        for k in range(_TPS):
            relx = (sx - xis[k]) + 3.0
            rely = (sy - yis[k]) + 3.0
            vx = relx.view(jnp.uint32) < SIX
            vy = rely.view(jnp.uint32) < SIX
            ne = (base + j_iota) != i_idss[k]
            valid = (vx & vy) & ne
            ci = (relx.astype(jnp.int32) << 3) | rely.astype(jnp.int32)
            v32 = jnp.int32(1) << (ci & 31)
            vv = jnp.where(valid, v32, 0)
            lo_add = jnp.where(ci < 32, vv, 0)
            accs[k][0] = accs[k][0] | lo_add
            accs[k][1] = accs[k][1] | (vv ^ lo_add)

    sh32 = jax.lax.broadcasted_iota(jnp.int32, (32, 1), 0)
    sh16 = jax.lax.broadcasted_iota(jnp.int32, (16, 1), 0)
    for k in range(_TPS):
        red = []
        for acc in accs[k]:
            a = acc
            s = _SJ
            while s > 1:
                h = s // 2
                a = a[:h] | a[h:s]
                s = h
            red.append(a)
        lo_b = jnp.broadcast_to(red[0], (32, _TI))
        hi_b = jnp.broadcast_to(red[1], (16, _TI))
        b_lo = (lo_b >> sh32) & 1
        b_hi = (hi_b >> sh16) & 1
        occT = jnp.concatenate([b_lo, b_hi], axis=0).astype(jnp.float32)

        out = jax.lax.dot_general(
            occT, w_ref[...],
            dimension_numbers=(((0,), (1,)), ((), ())),
            preferred_element_type=jnp.float32)
        out_ref[k * _TI:(k + 1) * _TI, :] = out + b_ref[...]


@functools.partial(jax.jit, static_argnames=())
def kernel(h, c, obs, W, b):
    del h, c
    obs2 = obs.astype(jnp.float32) * 2.0
    xj = jnp.broadcast_to(obs2[:, 0:1], (_N, _TI))
    yj = jnp.broadcast_to(obs2[:, 1:2], (_N, _TI))
    oiT = jnp.concatenate(
        [obs2.T, jnp.zeros((6, _N), jnp.float32)], axis=0)
    b2 = b.reshape(1, _HID).astype(jnp.float32)
    cols = jnp.array([bx * 8 + by for bx in range(_NG) for by in range(_NG)],
                     dtype=jnp.int32)
    W48 = jnp.zeros((_HID, 48), jnp.float32).at[:, cols].set(
        W.astype(jnp.float32))

    grid = (_N // (_TI * _TPS),)
    out = pl.pallas_call(
        _occ_kernel,
        grid=grid,
        in_specs=[
            pl.BlockSpec((_N, _TI), lambda t: (0, 0)),
            pl.BlockSpec((_N, _TI), lambda t: (0, 0)),
            pl.BlockSpec((8, _TPS * _TI), lambda t: (0, t)),
            pl.BlockSpec((_HID, 48), lambda t: (0, 0)),
            pl.BlockSpec((1, _HID), lambda t: (0, 0)),
        ],
        out_specs=pl.BlockSpec((_TPS * _TI, _HID), lambda t: (t, 0)),
        out_shape=jax.ShapeDtypeStruct((_N, _HID), jnp.float32),
        compiler_params=pltpu.CompilerParams(
            dimension_semantics=("parallel",)),
    )(xj, yj, oiT, W48, b2)
    return out
